# baseline (speedup 1.0000x reference)
import functools

import jax
import jax.numpy as jnp
from jax import lax
from jax.experimental import pallas as pl
from jax.experimental.pallas import tpu as pltpu
from jax.experimental.pallas import tpu_sc as plsc

_BATCH = 16384
_NUM_FIELDS_CAT = 26
_NUM_FIELDS_NUM = 13
_NW = 32
_NS = 16
_BPW = _BATCH // _NW
_LANES = 16

_SPF = 10
_HBF = _NUM_FIELDS_CAT - _SPF
_FIELD_ROWS = 100000
_SP_ROWS = _SPF * _FIELD_ROWS


_RCHUNK = 4000
_NCHUNKS = _SP_ROWS // _RCHUNK
_ROUNDS = (_NCHUNKS + _NS - 1) // _NS


def _sc_body(idxs_hbm, idxh_hbm, numx_hbm, wb_hbm, table_hbm, out_hbm,
             idxs_v, idxh_v, valss_v, valsh_v, numx_v, wb_v, out_v,
             bounce_v, sp_table, semh, sems):
  cid = lax.axis_index("c")
  sid = lax.axis_index("s")
  wid = sid * 2 + cid

  pltpu.sync_copy(idxh_hbm.at[wid], idxh_v)
  pltpu.sync_copy(idxs_hbm.at[wid], idxs_v)
  pltpu.sync_copy(numx_hbm.at[wid], numx_v)
  pltpu.sync_copy(wb_hbm, wb_v)

  hbm_cp = pltpu.async_copy(table_hbm.at[idxh_v], valsh_v, semh)

  def stage_round(r, carry):
    c = sid + r * _NS

    @pl.when(c < _NCHUNKS)
    def _():
      off = c * _RCHUNK
      pltpu.sync_copy(table_hbm.at[pl.ds(off, _RCHUNK)], bounce_v)
      pltpu.sync_copy(bounce_v, sp_table.at[pl.ds(off, _RCHUNK)])

    return carry

  lax.fori_loop(0, _ROUNDS, stage_round, 0)
  plsc.subcore_barrier()

  pltpu.async_copy(sp_table.at[idxs_v], valss_v, sems).wait()
  hbm_cp.wait()

  for g in range(_BPW // _LANES):
    col = g * _LANES

    def sp_step(f, acc, col=col):
      return acc + valss_v[pl.ds(f * _BPW + col, _LANES)]

    acc = lax.fori_loop(0, _SPF, sp_step, jnp.zeros((_LANES,), jnp.float32))

    def hb_step(f, acc, col=col):
      return acc + valsh_v[pl.ds(f * _BPW + col, _LANES)]

    acc = lax.fori_loop(0, _HBF, hb_step, acc)

    def num_step(f, acc, col=col):
      return acc + wb_v[f, :] * numx_v[f, pl.ds(col, _LANES)]

    acc = lax.fori_loop(0, _NUM_FIELDS_NUM, num_step, acc)
    out_v[pl.ds(col, _LANES)] = acc

  pltpu.sync_copy(out_v, out_hbm.at[pl.ds(wid * _BPW, _BPW)])


@jax.jit
def _run(idxs_arr, idxh_arr, numx_arr, wb_arr, table_flat):
  mesh = plsc.VectorSubcoreMesh(core_axis_name="c", subcore_axis_name="s",
                                num_cores=2, num_subcores=16)
  return pl.kernel(
      _sc_body,
      out_type=jax.ShapeDtypeStruct((_BATCH,), jnp.float32),
      mesh=mesh,
      scratch_types=[
          pltpu.VMEM((_SPF * _BPW,), jnp.int32),
          pltpu.VMEM((_HBF * _BPW,), jnp.int32),
          pltpu.VMEM((_SPF * _BPW,), jnp.float32),
          pltpu.VMEM((_HBF * _BPW,), jnp.float32),
          pltpu.VMEM((_NUM_FIELDS_NUM, _BPW), jnp.float32),
          pltpu.VMEM((_NUM_FIELDS_NUM, _LANES), jnp.float32),
          pltpu.VMEM((_BPW,), jnp.float32),
          pltpu.VMEM((_RCHUNK,), jnp.float32),
          pltpu.VMEM_SHARED((_SP_ROWS,), jnp.float32),
          pltpu.SemaphoreType.DMA,
          pltpu.SemaphoreType.DMA,
      ],
  )(idxs_arr, idxh_arr, numx_arr, wb_arr, table_flat)


def kernel(numerical_x, categorical_x, cat_table, num_weight, offsets):
  gidx = categorical_x + offsets[None, :]
  gidx = gidx.reshape(_NW, _BPW, _NUM_FIELDS_CAT)
  gidx = gidx.transpose(0, 2, 1)
  idxs_arr = gidx[:, :_SPF, :].reshape(_NW, _SPF * _BPW)
  idxh_arr = gidx[:, _SPF:, :].reshape(_NW, _HBF * _BPW)
  numx_arr = numerical_x.reshape(_NW, _BPW, _NUM_FIELDS_NUM)
  numx_arr = numx_arr.transpose(0, 2, 1)
  wb_arr = jnp.broadcast_to(num_weight.reshape(_NUM_FIELDS_NUM, 1),
                            (_NUM_FIELDS_NUM, _LANES))
  table_flat = cat_table.reshape(-1)
  return _run(idxs_arr, idxh_arr, numx_arr, wb_arr, table_flat)


# speedup vs baseline: 1.0284x; 1.0284x over previous
import functools

import jax
import jax.numpy as jnp
from jax import lax
from jax.experimental import pallas as pl
from jax.experimental.pallas import tpu as pltpu
from jax.experimental.pallas import tpu_sc as plsc

_BATCH = 16384
_NUM_FIELDS_CAT = 26
_NUM_FIELDS_NUM = 13
_NW = 32
_BPW = _BATCH // _NW
_CHUNK = 128
_NCHUNK = _BPW // _CHUNK
_LANES = 16


def _sc_body(idx_hbm, numx_hbm, wb_hbm, table_hbm, out_hbm,
             idx_v, vals_v, numx_v, wb_v, out_v, sem):
  cid = lax.axis_index("c")
  sid = lax.axis_index("s")
  wid = sid * 2 + cid

  pltpu.sync_copy(idx_hbm.at[wid], idx_v)
  pltpu.sync_copy(numx_hbm.at[wid], numx_v)
  pltpu.sync_copy(wb_hbm, wb_v)

  copies = []
  for c in range(_NCHUNK):
    for f in range(_NUM_FIELDS_CAT):
      copies.append(
          pltpu.async_copy(table_hbm.at[idx_v.at[c, f]], vals_v.at[c, f], sem))
  for cp in copies:
    cp.wait()

  for c in range(_NCHUNK):
    for g in range(_CHUNK // _LANES):
      col = c * _CHUNK + g * _LANES

      def cat_step(f, acc, c=c, g=g):
        return acc + vals_v[c, f, pl.ds(g * _LANES, _LANES)]

      acc = lax.fori_loop(0, _NUM_FIELDS_CAT, cat_step,
                          jnp.zeros((_LANES,), jnp.float32))

      def num_step(f, acc, col=col):
        return acc + wb_v[f, :] * numx_v[f, pl.ds(col, _LANES)]

      acc = lax.fori_loop(0, _NUM_FIELDS_NUM, num_step, acc)
      out_v[pl.ds(col, _LANES)] = acc

  pltpu.sync_copy(out_v, out_hbm.at[pl.ds(wid * _BPW, _BPW)])


@jax.jit
def _run(idx_arr, numx_arr, wb_arr, table_flat):
  mesh = plsc.VectorSubcoreMesh(core_axis_name="c", subcore_axis_name="s",
                                num_cores=2, num_subcores=16)
  return pl.kernel(
      _sc_body,
      out_type=jax.ShapeDtypeStruct((_BATCH,), jnp.float32),
      mesh=mesh,
      scratch_types=[
          pltpu.VMEM((_NCHUNK, _NUM_FIELDS_CAT, _CHUNK), jnp.int32),
          pltpu.VMEM((_NCHUNK, _NUM_FIELDS_CAT, _CHUNK), jnp.float32),
          pltpu.VMEM((_NUM_FIELDS_NUM, _BPW), jnp.float32),
          pltpu.VMEM((_NUM_FIELDS_NUM, _LANES), jnp.float32),
          pltpu.VMEM((_BPW,), jnp.float32),
          pltpu.SemaphoreType.DMA,
      ],
  )(idx_arr, numx_arr, wb_arr, table_flat)


def kernel(numerical_x, categorical_x, cat_table, num_weight, offsets):
  gidx = categorical_x + offsets[None, :]
  idx_arr = gidx.reshape(_NW, _NCHUNK, _CHUNK, _NUM_FIELDS_CAT)
  idx_arr = idx_arr.transpose(0, 1, 3, 2)
  numx_arr = numerical_x.reshape(_NW, _BPW, _NUM_FIELDS_NUM)
  numx_arr = numx_arr.transpose(0, 2, 1)
  wb_arr = jnp.broadcast_to(num_weight.reshape(_NUM_FIELDS_NUM, 1),
                            (_NUM_FIELDS_NUM, _LANES))
  table_flat = cat_table.reshape(-1)
  return _run(idx_arr, numx_arr, wb_arr, table_flat)


# speedup vs baseline: 1.0367x; 1.0080x over previous
import functools

import jax
import jax.numpy as jnp
from jax import lax
from jax.experimental import pallas as pl
from jax.experimental.pallas import tpu as pltpu
from jax.experimental.pallas import tpu_sc as plsc

_BATCH = 16384
_NUM_FIELDS_CAT = 26
_NUM_FIELDS_NUM = 13
_NW = 32
_BPW = _BATCH // _NW
_CHUNK = 128
_NCHUNK = _BPW // _CHUNK
_LANES = 16


def _sc_body(idx_hbm, numx_hbm, wb_hbm, table_hbm, out_hbm,
             idx_v, vals_v, numx_v, wb_v, out_v, sem):
  cid = lax.axis_index("c")
  sid = lax.axis_index("s")
  wid = sid * 2 + cid

  pltpu.sync_copy(idx_hbm.at[wid], idx_v)
  pltpu.sync_copy(numx_hbm.at[wid], numx_v)
  pltpu.sync_copy(wb_hbm, wb_v)

  cp = pltpu.async_copy(table_hbm.at[idx_v], vals_v, sem)
  cp.wait()

  for c in range(_NCHUNK):
    for g in range(_CHUNK // _LANES):
      col = c * _CHUNK + g * _LANES

      def cat_step(f, acc, c=c, g=g):
        return acc + vals_v[pl.ds((c * _NUM_FIELDS_CAT + f) * _CHUNK
                                  + g * _LANES, _LANES)]

      acc = lax.fori_loop(0, _NUM_FIELDS_CAT, cat_step,
                          jnp.zeros((_LANES,), jnp.float32))

      def num_step(f, acc, col=col):
        return acc + wb_v[f, :] * numx_v[f, pl.ds(col, _LANES)]

      acc = lax.fori_loop(0, _NUM_FIELDS_NUM, num_step, acc)
      out_v[pl.ds(col, _LANES)] = acc

  pltpu.sync_copy(out_v, out_hbm.at[pl.ds(wid * _BPW, _BPW)])


@jax.jit
def _run(idx_arr, numx_arr, wb_arr, table_flat):
  mesh = plsc.VectorSubcoreMesh(core_axis_name="c", subcore_axis_name="s",
                                num_cores=2, num_subcores=16)
  return pl.kernel(
      _sc_body,
      out_type=jax.ShapeDtypeStruct((_BATCH,), jnp.float32),
      mesh=mesh,
      scratch_types=[
          pltpu.VMEM((_NCHUNK * _NUM_FIELDS_CAT * _CHUNK,), jnp.int32),
          pltpu.VMEM((_NCHUNK * _NUM_FIELDS_CAT * _CHUNK,), jnp.float32),
          pltpu.VMEM((_NUM_FIELDS_NUM, _BPW), jnp.float32),
          pltpu.VMEM((_NUM_FIELDS_NUM, _LANES), jnp.float32),
          pltpu.VMEM((_BPW,), jnp.float32),
          pltpu.SemaphoreType.DMA,
      ],
  )(idx_arr, numx_arr, wb_arr, table_flat)


def kernel(numerical_x, categorical_x, cat_table, num_weight, offsets):
  gidx = categorical_x + offsets[None, :]
  idx_arr = gidx.reshape(_NW, _NCHUNK, _CHUNK, _NUM_FIELDS_CAT)
  idx_arr = idx_arr.transpose(0, 1, 3, 2)
  idx_arr = idx_arr.reshape(_NW, -1)
  numx_arr = numerical_x.reshape(_NW, _BPW, _NUM_FIELDS_NUM)
  numx_arr = numx_arr.transpose(0, 2, 1)
  wb_arr = jnp.broadcast_to(num_weight.reshape(_NUM_FIELDS_NUM, 1),
                            (_NUM_FIELDS_NUM, _LANES))
  table_flat = cat_table.reshape(-1)
  return _run(idx_arr, numx_arr, wb_arr, table_flat)


# speedup vs baseline: 1.0428x; 1.0059x over previous
import functools

import jax
import jax.numpy as jnp
from jax import lax
from jax.experimental import pallas as pl
from jax.experimental.pallas import tpu as pltpu
from jax.experimental.pallas import tpu_sc as plsc

_BATCH = 16384
_NUM_FIELDS_CAT = 26
_NUM_FIELDS_NUM = 13
_NW = 32
_BPW = _BATCH // _NW
_CHUNK = 128
_NCHUNK = _BPW // _CHUNK
_LANES = 16


def _sc_body(idx_hbm, numx_hbm, wb_hbm, table_hbm, out_hbm,
             idx_v, vals_v, numx_v, wb_v, out_v, sem):
  cid = lax.axis_index("c")
  sid = lax.axis_index("s")
  wid = sid * 2 + cid

  pltpu.sync_copy(idx_hbm.at[wid], idx_v)
  pltpu.sync_copy(numx_hbm.at[wid], numx_v)
  pltpu.sync_copy(wb_hbm, wb_v)

  cp = pltpu.async_copy(table_hbm.at[idx_v], vals_v, sem)
  cp.wait()

  for c in range(_NCHUNK):
    for g in range(_CHUNK // _LANES):
      col = c * _CHUNK + g * _LANES

      def cat_step(f, acc, col=col):
        return acc + vals_v[pl.ds(f * _BPW + col, _LANES)]

      acc = lax.fori_loop(0, _NUM_FIELDS_CAT, cat_step,
                          jnp.zeros((_LANES,), jnp.float32))

      def num_step(f, acc, col=col):
        return acc + wb_v[f, :] * numx_v[f, pl.ds(col, _LANES)]

      acc = lax.fori_loop(0, _NUM_FIELDS_NUM, num_step, acc)
      out_v[pl.ds(col, _LANES)] = acc

  pltpu.sync_copy(out_v, out_hbm.at[pl.ds(wid * _BPW, _BPW)])


@jax.jit
def _run(idx_arr, numx_arr, wb_arr, table_flat):
  mesh = plsc.VectorSubcoreMesh(core_axis_name="c", subcore_axis_name="s",
                                num_cores=2, num_subcores=16)
  return pl.kernel(
      _sc_body,
      out_type=jax.ShapeDtypeStruct((_BATCH,), jnp.float32),
      mesh=mesh,
      scratch_types=[
          pltpu.VMEM((_NCHUNK * _NUM_FIELDS_CAT * _CHUNK,), jnp.int32),
          pltpu.VMEM((_NCHUNK * _NUM_FIELDS_CAT * _CHUNK,), jnp.float32),
          pltpu.VMEM((_NUM_FIELDS_NUM, _BPW), jnp.float32),
          pltpu.VMEM((_NUM_FIELDS_NUM, _LANES), jnp.float32),
          pltpu.VMEM((_BPW,), jnp.float32),
          pltpu.SemaphoreType.DMA,
      ],
  )(idx_arr, numx_arr, wb_arr, table_flat)


def kernel(numerical_x, categorical_x, cat_table, num_weight, offsets):
  gidx = categorical_x + offsets[None, :]
  idx_arr = gidx.reshape(_NW, _BPW, _NUM_FIELDS_CAT)
  idx_arr = idx_arr.transpose(0, 2, 1)
  idx_arr = idx_arr.reshape(_NW, -1)
  numx_arr = numerical_x.reshape(_NW, _BPW, _NUM_FIELDS_NUM)
  numx_arr = numx_arr.transpose(0, 2, 1)
  wb_arr = jnp.broadcast_to(num_weight.reshape(_NUM_FIELDS_NUM, 1),
                            (_NUM_FIELDS_NUM, _LANES))
  table_flat = cat_table.reshape(-1)
  return _run(idx_arr, numx_arr, wb_arr, table_flat)
